# baseline (speedup 1.0000x reference)
import functools

import jax
import jax.numpy as jnp
from jax import lax
from jax.experimental import pallas as pl
from jax.experimental.pallas import tpu as pltpu
from jax.experimental.pallas import tpu_sc as plsc

_EPS = 1e-12


def _make_sc_gather(tok, hidden, chunk):
    info = plsc.get_sparse_core_info()
    nc, ns = info.num_cores, info.num_subcores
    nw = nc * ns
    per_w = tok // nw
    nch = per_w // chunk

    mesh = plsc.VectorSubcoreMesh(core_axis_name="c", subcore_axis_name="s")

    @functools.partial(
        pl.kernel,
        mesh=mesh,
        out_type=jax.ShapeDtypeStruct((tok, hidden), jnp.float32),
        scratch_types=[
            pltpu.VMEM((nch, chunk), jnp.int32),
            pltpu.VMEM((chunk, hidden), jnp.float32),
            pltpu.VMEM((chunk, hidden), jnp.float32),
            pltpu.SemaphoreType.DMA,
            pltpu.SemaphoreType.DMA,
        ],
    )
    def gather_kernel(table_hbm, idx_hbm, out_hbm, idx_v,
                      buf0, buf1, gsem0, gsem1):
        wid = lax.axis_index("s") * nc + lax.axis_index("c")
        base = wid * per_w
        pltpu.sync_copy(idx_hbm.at[wid], idx_v)
        bufs = (buf0, buf1)
        gsems = (gsem0, gsem1)
        gh = [pltpu.async_copy(table_hbm.at[idx_v.at[0]], buf0, gsem0), None]
        for c in range(nch):
            cur = c % 2
            nxt = (c + 1) % 2
            if c + 1 < nch:
                gh[nxt] = pltpu.async_copy(
                    table_hbm.at[idx_v.at[c + 1]], bufs[nxt], gsems[nxt])
            gh[cur].wait()
            pltpu.sync_copy(bufs[cur], out_hbm.at[pl.ds(base + c * chunk, chunk)])

    return gather_kernel


def _ln_compute(dids_ref, g_ref, pos_ref, dom_ref, gam_ref, bet_ref, out_ref):
    tb, hidden = g_ref.shape
    pr = pos_ref.shape[0]
    x = (g_ref[...].reshape(tb // pr, pr, hidden)
         + pos_ref[...][None]).reshape(tb, hidden)
    ids = dids_ref[...].astype(jnp.int32)
    oh = (ids == lax.broadcasted_iota(jnp.int32, (ids.shape[0], 16), 1))
    x = x + jnp.dot(oh.astype(jnp.float32), dom_ref[...],
                    preferred_element_type=jnp.float32)
    mean = jnp.mean(x, axis=-1, keepdims=True)
    xc = x - mean
    var = jnp.mean(xc * xc, axis=-1, keepdims=True)
    out_ref[...] = xc * lax.rsqrt(var + _EPS) * gam_ref[...] + bet_ref[...]


def _make_tc_ln_stage(tok, hidden, tb, stage_tok, blk0, first):
    grid = stage_tok // tb

    common_in_specs = [
        pl.BlockSpec((tb, 1), lambda i: (blk0 + i, 0)),
        pl.BlockSpec((tb, hidden), lambda i: (i, 0)),
        pl.BlockSpec((512, hidden), lambda i: (0, 0)),
        pl.BlockSpec((16, hidden), lambda i: (0, 0)),
        pl.BlockSpec((1, hidden), lambda i: (0, 0)),
        pl.BlockSpec((1, hidden), lambda i: (0, 0)),
    ]
    out_spec = pl.BlockSpec((tb, hidden), lambda i: (blk0 + i, 0))
    out_shape = jax.ShapeDtypeStruct((tok, hidden), jnp.float32)

    if first:
        return pl.pallas_call(
            _ln_compute,
            grid=(grid,),
            in_specs=common_in_specs,
            out_specs=out_spec,
            out_shape=out_shape,
        )

    def body(prev_ref, dids_ref, g_ref, pos_ref, dom_ref, gam_ref, bet_ref,
             out_ref):
        del prev_ref
        _ln_compute(dids_ref, g_ref, pos_ref, dom_ref, gam_ref, bet_ref,
                    out_ref)

    return pl.pallas_call(
        body,
        grid=(grid,),
        in_specs=[pl.BlockSpec(memory_space=pl.ANY)] + common_in_specs,
        out_specs=out_spec,
        out_shape=out_shape,
        input_output_aliases={0: 0},
    )


@jax.jit
def kernel(input_ids, domain_ids, word_table, pos_table, dom_table, gamma, beta):
    b, s = input_ids.shape
    hidden = word_table.shape[1]
    tok = b * s
    chunk = 64
    tb = 1024
    stage_toks = (tok,)

    idx2d = input_ids.astype(jnp.int32).reshape(tok // chunk, chunk)
    gathered = []
    row0 = 0
    nw = 32
    for st in stage_toks:
        rows = st // chunk
        idx3d = lax.slice_in_dim(idx2d, row0, row0 + rows).reshape(
            nw, rows // nw, chunk)
        gathered.append(_make_sc_gather(st, hidden, chunk)(word_table, idx3d))
        row0 += rows

    dids = domain_ids.astype(jnp.int8).reshape(tok, 1)
    pos_tiled = pos_table
    dom_pad = jnp.zeros((16, hidden), jnp.float32).at[: dom_table.shape[0]].set(dom_table)
    gam = gamma.reshape(1, hidden)
    bet = beta.reshape(1, hidden)

    out = None
    blk0 = 0
    for i, st in enumerate(stage_toks):
        ln = _make_tc_ln_stage(tok, hidden, tb, st, blk0, first=(i == 0))
        if i == 0:
            out = ln(dids, gathered[i], pos_tiled, dom_pad, gam, bet)
        else:
            out = ln(out, dids, gathered[i], pos_tiled, dom_pad, gam, bet)
        blk0 += st // tb
    return out.reshape(b, s, hidden)


# speedup vs baseline: 1.0512x; 1.0512x over previous
import functools

import jax
import jax.numpy as jnp
from jax import lax
from jax.experimental import pallas as pl
from jax.experimental.pallas import tpu as pltpu
from jax.experimental.pallas import tpu_sc as plsc

_EPS = 1e-12


def _make_sc_gather(tok, hidden, chunk):
    info = plsc.get_sparse_core_info()
    nc, ns = info.num_cores, info.num_subcores
    nw = nc * ns
    per_w = tok // nw
    nch = per_w // chunk

    mesh = plsc.VectorSubcoreMesh(core_axis_name="c", subcore_axis_name="s")

    @functools.partial(
        pl.kernel,
        mesh=mesh,
        out_type=jax.ShapeDtypeStruct((tok, hidden), jnp.float32),
        scratch_types=[
            pltpu.VMEM((nch, chunk), jnp.int32),
            pltpu.VMEM((chunk, hidden), jnp.float32),
            pltpu.VMEM((chunk, hidden), jnp.float32),
            pltpu.SemaphoreType.DMA,
            pltpu.SemaphoreType.DMA,
        ],
    )
    def gather_kernel(table_hbm, idx_hbm, out_hbm, idx_v,
                      buf0, buf1, gsem0, gsem1):
        wid = lax.axis_index("s") * nc + lax.axis_index("c")
        base = wid * per_w
        pltpu.sync_copy(idx_hbm.at[wid], idx_v)
        bufs = (buf0, buf1)
        gsems = (gsem0, gsem1)
        gh = [pltpu.async_copy(table_hbm.at[idx_v.at[0]], buf0, gsem0), None]
        for c in range(nch):
            cur = c % 2
            nxt = (c + 1) % 2
            if c + 1 < nch:
                gh[nxt] = pltpu.async_copy(
                    table_hbm.at[idx_v.at[c + 1]], bufs[nxt], gsems[nxt])
            gh[cur].wait()
            pltpu.sync_copy(bufs[cur], out_hbm.at[pl.ds(base + c * chunk, chunk)])

    return gather_kernel


def _ln_compute(dids_ref, g_ref, pos_ref, dom_ref, gam_ref, bet_ref, out_ref):
    tb, hidden = g_ref.shape
    pr = pos_ref.shape[0]
    x = (g_ref[...].reshape(tb // pr, pr, hidden)
         + pos_ref[...][None]).reshape(tb, hidden)
    ids = dids_ref[...].astype(jnp.int32)
    oh = (ids == lax.broadcasted_iota(jnp.int32, (ids.shape[0], 16), 1))
    x = x + jnp.dot(oh.astype(jnp.float32), dom_ref[...],
                    preferred_element_type=jnp.float32)
    mean = jnp.mean(x, axis=-1, keepdims=True)
    xc = x - mean
    var = jnp.mean(xc * xc, axis=-1, keepdims=True)
    out_ref[...] = xc * lax.rsqrt(var + _EPS) * gam_ref[...] + bet_ref[...]


def _make_tc_ln_stage(tok, hidden, tb, stage_tok, blk0, first):
    grid = stage_tok // tb

    common_in_specs = [
        pl.BlockSpec((tb, 1), lambda i: (blk0 + i, 0)),
        pl.BlockSpec((tb, hidden), lambda i: (i, 0)),
        pl.BlockSpec((512, hidden), lambda i: (0, 0)),
        pl.BlockSpec((16, hidden), lambda i: (0, 0)),
        pl.BlockSpec((1, hidden), lambda i: (0, 0)),
        pl.BlockSpec((1, hidden), lambda i: (0, 0)),
    ]
    out_spec = pl.BlockSpec((tb, hidden), lambda i: (blk0 + i, 0))
    out_shape = jax.ShapeDtypeStruct((tok, hidden), jnp.float32)

    if first:
        return pl.pallas_call(
            _ln_compute,
            grid=(grid,),
            in_specs=common_in_specs,
            out_specs=out_spec,
            out_shape=out_shape,
        )

    def body(prev_ref, dids_ref, g_ref, pos_ref, dom_ref, gam_ref, bet_ref,
             out_ref):
        del prev_ref
        _ln_compute(dids_ref, g_ref, pos_ref, dom_ref, gam_ref, bet_ref,
                    out_ref)

    return pl.pallas_call(
        body,
        grid=(grid,),
        in_specs=[pl.BlockSpec(memory_space=pl.ANY)] + common_in_specs,
        out_specs=out_spec,
        out_shape=out_shape,
        input_output_aliases={0: 0},
    )


@jax.jit
def kernel(input_ids, domain_ids, word_table, pos_table, dom_table, gamma, beta):
    b, s = input_ids.shape
    hidden = word_table.shape[1]
    tok = b * s
    chunk = 64
    tb = 2048
    stage_toks = (tok,)

    idx2d = input_ids.astype(jnp.int32).reshape(tok // chunk, chunk)
    gathered = []
    row0 = 0
    nw = 32
    for st in stage_toks:
        rows = st // chunk
        idx3d = lax.slice_in_dim(idx2d, row0, row0 + rows).reshape(
            nw, rows // nw, chunk)
        gathered.append(_make_sc_gather(st, hidden, chunk)(word_table, idx3d))
        row0 += rows

    dids = domain_ids.astype(jnp.int8).reshape(tok, 1)
    pos_tiled = pos_table
    dom_pad = jnp.zeros((16, hidden), jnp.float32).at[: dom_table.shape[0]].set(dom_table)
    gam = gamma.reshape(1, hidden)
    bet = beta.reshape(1, hidden)

    out = None
    blk0 = 0
    for i, st in enumerate(stage_toks):
        ln = _make_tc_ln_stage(tok, hidden, tb, st, blk0, first=(i == 0))
        if i == 0:
            out = ln(dids, gathered[i], pos_tiled, dom_pad, gam, bet)
        else:
            out = ln(out, dids, gathered[i], pos_tiled, dom_pad, gam, bet)
        blk0 += st // tb
    return out.reshape(b, s, hidden)
